# baseline (speedup 1.0000x reference)
import functools

import jax
import jax.numpy as jnp
import numpy as np
from jax import lax
from jax.experimental import pallas as pl
from jax.experimental.pallas import tpu as pltpu
from jax.experimental.pallas import tpu_sc as plsc

_B = 16384
_NCAT = 26
_D = 32
_V = 2600000
_NUM = 13
_OFFSETS = np.arange(_NCAT, dtype=np.int32) * 100000

_NW = 32
_IPR = 128
_N_ROWS = _B * _NCAT // _IPR
_ROWS_PER_W = _N_ROWS // _NW
_GPS = 8
_STEP = _GPS * _IPR
_NSTEP = _ROWS_PER_W // _GPS

_TW = 512
_NBLK = _V // _TW
_TAIL = _V - _NBLK * _TW


def _wid():
  return lax.axis_index("s") * 2 + lax.axis_index("c")


def _repack(table_t, tail128):
  mesh = plsc.VectorSubcoreMesh(core_axis_name="c", subcore_axis_name="s")

  @functools.partial(
      pl.kernel,
      mesh=mesh,
      compiler_params=pltpu.CompilerParams(needs_layout_passes=False),
      out_type=jax.ShapeDtypeStruct((_V // 4, _IPR), jnp.float32),
      scratch_types=[
          pltpu.VMEM((2, _D, _TW), jnp.float32),
          pltpu.VMEM((2, _TW // 4, _IPR), jnp.float32),
          pltpu.VMEM((_TAIL // 4, _IPR), jnp.float32),
          pltpu.SemaphoreType.DMA((2,)),
          pltpu.SemaphoreType.DMA((2,)),
      ],
      name="emb_repack",
  )
  def k(tab_hbm, tail_hbm, out_hbm, src_v, dst_v, tail_v, isem, osem):
    w = _wid()
    row_lo = lax.iota(jnp.int32, 16)
    row_hi = row_lo + 16

    def transpose_block(src, s, nrows):
      lane16 = lax.iota(jnp.int32, 16)
      row_pat = lane16 // 4
      lane_pat = (lane16 % 4) * 32
      dst2 = dst_v.at[s]
      for d in range(_D):
        lanes = lane_pat + d

        @plsc.parallel_loop(0, nrows // 4, unroll=8)
        def _(i):
          vals = src[d, pl.ds(i * 16, 16)]
          rows = row_pat + 4 * i
          plsc.store_scatter(dst2, [rows, lanes], vals)

    def fire(b, s):
      return pltpu.async_copy(
          tab_hbm.at[:, pl.ds(b * _TW, _TW)], src_v.at[s], isem.at[s]
      )

    def flush(b, s):
      return pltpu.async_copy(
          dst_v.at[s], out_hbm.at[pl.ds(b * (_TW // 4), _TW // 4)], osem.at[s]
      )

    nblocks = _NBLK // _NW
    nextra = _NBLK - nblocks * _NW

    def blk(i):
      return w + i * _NW

    fire(blk(0), 0).wait()

    @pl.loop(0, nblocks, step=2)
    def _(i):
      for s in range(2):
        @pl.when(i + s + 1 < nblocks)
        def _():
          fire(blk(i + s + 1), 1 - s).wait()

        transpose_block(src_v.at[s], s, _TW // 4)
        flush(blk(i + s), s).wait()

    @pl.when(w < nextra)
    def _():
      fire(blk(nblocks), 0).wait()
      transpose_block(src_v.at[0], 0, _TW // 4)
      flush(blk(nblocks), 0).wait()

    @pl.when(w == _NW - 1)
    def _():
      pltpu.sync_copy(tail_hbm, tail_v)
      pltpu.sync_copy(tail_v, out_hbm.at[pl.ds(_NBLK * (_TW // 4), _TAIL // 4)])

  return k(table_t, tail128)


def _gather(t128, idx):
  mesh = plsc.VectorSubcoreMesh(core_axis_name="c", subcore_axis_name="s")

  @functools.partial(
      pl.kernel,
      mesh=mesh,
      compiler_params=pltpu.CompilerParams(use_tc_tiling_on_sc=False),
      out_type=jax.ShapeDtypeStruct((_B * _NCAT, _D), jnp.float32),
      scratch_types=[
          pltpu.VMEM((_ROWS_PER_W, _IPR), jnp.int32),
          pltpu.VMEM((2, _STEP, _D), jnp.float32),
          pltpu.SemaphoreType.DMA((2,)),
          pltpu.SemaphoreType.DMA((2,)),
      ],
      name="emb_gather",
  )
  def k(table_hbm, idx_hbm, out_hbm, idx_v, rows_v, gsem, ssem):
    w = _wid()
    idx_base = w * _ROWS_PER_W
    out_base = w * _ROWS_PER_W * _IPR

    pltpu.sync_copy(idx_hbm.at[pl.ds(idx_base, _ROWS_PER_W)], idx_v)

    def fire(i):
      s = i % 2
      return [
          pltpu.async_copy(
              table_hbm.at[idx_v.at[i * _GPS + j]],
              rows_v.at[s, pl.ds(j * _IPR, _IPR)],
              gsem.at[s],
          )
          for j in range(_GPS)
      ]

    g_descs = [None] * _NSTEP
    s_descs = [None] * _NSTEP
    g_descs[0] = fire(0)
    for i in range(_NSTEP):
      s = i % 2
      if i + 1 < _NSTEP:
        if i >= 1:
          s_descs[i - 1].wait()
        g_descs[i + 1] = fire(i + 1)
      for d in g_descs[i]:
        d.wait()
      s_descs[i] = pltpu.async_copy(
          rows_v.at[s], out_hbm.at[pl.ds(out_base + i * _STEP, _STEP)], ssem.at[s]
      )
    s_descs[_NSTEP - 2].wait()
    s_descs[_NSTEP - 1].wait()

  return k(t128, idx)


def kernel(x, table):
  idx = (x[:, _NUM:].astype(jnp.int32) + _OFFSETS[None, :]).reshape(_N_ROWS, _IPR)
  tail128 = table[_NBLK * _TW : _V].reshape(_TAIL // 4, _IPR)
  t128 = _repack(table.T, tail128)
  emb = _gather(t128.reshape(_V, _D), idx)
  return jnp.concatenate([x[:, :_NUM], emb.reshape(_B, _NCAT * _D)], axis=1)


# speedup vs baseline: 1.4083x; 1.4083x over previous
import functools

import jax
import jax.numpy as jnp
import numpy as np
from jax import lax
from jax.experimental import pallas as pl
from jax.experimental.pallas import tpu as pltpu
from jax.experimental.pallas import tpu_sc as plsc

_B = 16384
_NCAT = 26
_D = 32
_NUM = 13
_OFFSETS = np.arange(_NCAT, dtype=np.int32) * 100000

_NW = 32
_IPR = 128
_N_ROWS = _B * _NCAT // _IPR
_ROWS_PER_W = _N_ROWS // _NW
_GPS = 8
_STEP = _GPS * _IPR
_NSTEP = _ROWS_PER_W // _GPS


def _emb_gather(table, idx):
  mesh = plsc.VectorSubcoreMesh(core_axis_name="c", subcore_axis_name="s")

  @functools.partial(
      pl.kernel,
      mesh=mesh,
      compiler_params=pltpu.CompilerParams(use_tc_tiling_on_sc=False),
      out_type=jax.ShapeDtypeStruct((_B * _NCAT, _D), jnp.float32),
      scratch_types=[
          pltpu.VMEM((_ROWS_PER_W, _IPR), jnp.int32),
          pltpu.VMEM((2, _STEP, _D), jnp.float32),
          pltpu.SemaphoreType.DMA((2,)),
          pltpu.SemaphoreType.DMA((2,)),
      ],
      name="emb_gather",
  )
  def k(table_hbm, idx_hbm, out_hbm, idx_v, rows_v, gsem, ssem):
    w = lax.axis_index("s") * 2 + lax.axis_index("c")
    idx_base = w * _ROWS_PER_W
    out_base = w * _ROWS_PER_W * _IPR

    pltpu.sync_copy(idx_hbm.at[pl.ds(idx_base, _ROWS_PER_W)], idx_v)

    def fire(i):
      s = i % 2
      return [
          pltpu.async_copy(
              table_hbm.at[idx_v.at[i * _GPS + j]],
              rows_v.at[s, pl.ds(j * _IPR, _IPR)],
              gsem.at[s],
          )
          for j in range(_GPS)
      ]

    g_descs = [None] * _NSTEP
    s_descs = [None] * _NSTEP
    g_descs[0] = fire(0)
    for i in range(_NSTEP):
      s = i % 2
      if i + 1 < _NSTEP:
        if i >= 1:
          s_descs[i - 1].wait()
        g_descs[i + 1] = fire(i + 1)
      for d in g_descs[i]:
        d.wait()
      s_descs[i] = pltpu.async_copy(
          rows_v.at[s], out_hbm.at[pl.ds(out_base + i * _STEP, _STEP)], ssem.at[s]
      )
    s_descs[_NSTEP - 2].wait()
    s_descs[_NSTEP - 1].wait()

  return k(table, idx)


def kernel(x, table):
  idx = (x[:, _NUM:].astype(jnp.int32) + _OFFSETS[None, :]).reshape(_N_ROWS, _IPR)
  emb = _emb_gather(table, idx)
  return jnp.concatenate([x[:, :_NUM], emb.reshape(_B, _NCAT * _D)], axis=1)
